# baseline (speedup 1.0000x reference)
import jax
import jax.numpy as jnp
from jax.experimental import pallas as pl

B, D, K_NEG, N_HARD = 1024, 256, 65536, 1024
N1, N2, N3 = 128, 128, 128
TEMP = 0.2
HARD_ALPHA = 0.5
HARD_BETA = 1.5
HARD_GAMMA = 1.0

_RB, _CB = 256, 512


def _mm_body(q_ref, kq_ref, o_ref, qn_out_ref):
    kq = kq_ref[...]
    norm = jnp.sqrt(jnp.sum(kq * kq, axis=0, keepdims=True))
    kqn = kq / jnp.clip(norm, 1e-12, None)
    qn_out_ref[...] = kqn
    o_ref[...] = jnp.dot(q_ref[...], kqn,
                         preferred_element_type=jnp.float32) * (1.0 / TEMP)


def _logits_scaled(q_n, queue):
    return pl.pallas_call(
        _mm_body,
        grid=(B // _RB, K_NEG // _CB),
        in_specs=[
            pl.BlockSpec((_RB, D), lambda i, j: (i, 0)),
            pl.BlockSpec((D, _CB), lambda i, j: (0, j)),
        ],
        out_specs=[
            pl.BlockSpec((_RB, _CB), lambda i, j: (i, j)),
            pl.BlockSpec((D, _CB), lambda i, j: (0, j)),
        ],
        out_shape=[
            jax.ShapeDtypeStruct((B, K_NEG), jnp.float32),
            jax.ShapeDtypeStruct((D, K_NEG), jnp.float32),
        ],
    )(q_n, queue)


def kernel(q, queue):
    qn = q / jnp.clip(jnp.linalg.norm(q, axis=-1, keepdims=True), 1e-12, None)
    logits_s, queue_n = _logits_scaled(qn, queue)

    _, idxs_hard = jax.lax.top_k(logits_s, N_HARD)
    rk = jax.random.key(42)
    k_i1, k_a1, k_i2, k_b2, k_i3a, k_i3b, k_g3 = jax.random.split(rk, 7)
    queue_t = queue_n.T
    idxs1 = jax.random.randint(k_i1, (B, N1), 0, N_HARD)
    alpha = jax.random.uniform(k_a1, (B, N1, 1), dtype=jnp.float32) * HARD_ALPHA
    sel1 = jnp.take_along_axis(idxs_hard, idxs1, axis=1)
    hn1 = queue_t[sel1]
    s1 = alpha * qn[:, None, :] + (1.0 - alpha) * hn1
    s1 = s1 / jnp.clip(jnp.linalg.norm(s1, axis=-1, keepdims=True), 1e-12, None)
    idxs2 = jax.random.randint(k_i2, (B, N2), 0, N_HARD)
    beta = 1.0 + jax.random.uniform(k_b2, (B, N2, 1), dtype=jnp.float32) * (HARD_BETA - 1.0)
    sel2 = jnp.take_along_axis(idxs_hard, idxs2, axis=1)
    hn2 = queue_t[sel2]
    s2 = qn[:, None, :] + beta * (hn2 - qn[:, None, :])
    s2 = s2 / jnp.clip(jnp.linalg.norm(s2, axis=-1, keepdims=True), 1e-12, None)
    idxs3a = jax.random.randint(k_i3a, (B, N3), 0, N_HARD)
    idxs3b = jax.random.randint(k_i3b, (B, N3), 0, N_HARD)
    gamma = jax.random.uniform(k_g3, (B, N3, 1), dtype=jnp.float32) * HARD_GAMMA
    hn3a = queue_t[jnp.take_along_axis(idxs_hard, idxs3a, axis=1)]
    hn3b = queue_t[jnp.take_along_axis(idxs_hard, idxs3b, axis=1)]
    s3 = gamma * hn3a + (1.0 - gamma) * hn3b
    s3 = s3 / jnp.clip(jnp.linalg.norm(s3, axis=-1, keepdims=True), 1e-12, None)
    neg = jnp.concatenate([s1, s2, s3], axis=1)
    l_hard = jnp.einsum('bd,bnd->bn', qn, neg) * (1.0 / TEMP)
    return jnp.concatenate([logits_s, l_hard], axis=1)
